# Initial kernel scaffold; baseline (speedup 1.0000x reference)
import jax
import jax.numpy as jnp
from jax.experimental import pallas as pl


def kernel(x, edge_index, W0, b0, g0, beta0, W1, b1, g1, beta1, W2, b2, g2, beta2):
    raise NotImplementedError("write your pallas kernel here")


# speedup vs baseline: 3.7732x; 3.7732x over previous
import functools

import jax
import jax.numpy as jnp
from jax import lax
from jax.experimental import pallas as pl
from jax.experimental.pallas import tpu as pltpu
from jax.experimental.pallas import tpu_sc as plsc

N_NODES = 10000
N_EDGES = 320000
IN_DIM = 128
HID = 256
LN_EPS = 1e-5

NC = 2
NS = 16
CHUNK = 128
N_ACC = 10112
DUMMY_ROW = N_NODES
ZROWS = N_ACC // NS


def _pad_per_tile(n_per_tile: int) -> int:
    return ((n_per_tile + CHUNK - 1) // CHUNK) * CHUNK


def _make_sc_aggregate(e_pt: int):
    nchunks = e_pt // CHUNK
    mesh = plsc.VectorSubcoreMesh(
        core_axis_name="c", subcore_axis_name="s", num_cores=NC, num_subcores=NS)

    @functools.partial(
        pl.kernel,
        mesh=mesh,
        out_type=jax.ShapeDtypeStruct((NC * N_ACC, 128), jnp.float32),
        scratch_types=[
            pltpu.VMEM((CHUNK,), jnp.int32),
            pltpu.VMEM((CHUNK,), jnp.int32),
            pltpu.VMEM((CHUNK, 128), jnp.float32),
            pltpu.VMEM_SHARED((N_ACC, 128), jnp.float32),
            pltpu.SemaphoreType.DMA,
        ],
    )
    def k(xv_hbm, gidx_hbm, ddst_hbm, out_hbm, ibuf, dbuf, rows, acc, sem):
        c = lax.axis_index("c")
        s = lax.axis_index("s")
        zero16 = jnp.zeros((16,), jnp.float32)

        def zrow(r, carry):
            for j in range(8):
                rows[r, pl.ds(j * 16, 16)] = zero16
            return carry

        lax.fori_loop(0, CHUNK, zrow, 0)
        for kk in range(5):
            sz = 128 if kk < 4 else ZROWS - 4 * 128
            pltpu.sync_copy(rows.at[pl.ds(0, sz)], acc.at[pl.ds(s * ZROWS + kk * 128, sz)])
        plsc.subcore_barrier()

        tile_base = (c * NS + s) * e_pt

        def chunk_body(g, carry):
            base = tile_base + g * CHUNK
            pltpu.sync_copy(gidx_hbm.at[pl.ds(base, CHUNK)], ibuf)
            pltpu.sync_copy(ddst_hbm.at[pl.ds(base, CHUNK)], dbuf)
            pltpu.async_copy(xv_hbm.at[ibuf], rows, sem).wait()
            pltpu.sync_copy(rows, acc.at[dbuf], add=True)
            return carry

        lax.fori_loop(0, nchunks, chunk_body, 0)
        plsc.subcore_barrier()

        n_out_chunks = N_ACC // 128

        def copy_chunk(j):
            @pl.when(j < n_out_chunks)
            def _():
                pltpu.sync_copy(acc.at[pl.ds(j * 128, 128)], rows)
                pltpu.sync_copy(rows, out_hbm.at[pl.ds(c * N_ACC + j * 128, 128)])

        for t in range(5):
            copy_chunk(s + NS * t)

    return k


_sc_agg_cache = {}


def _sc_aggregate(xv, gidx, ddst, e_pt):
    if e_pt not in _sc_agg_cache:
        _sc_agg_cache[e_pt] = _make_sc_aggregate(e_pt)
    out = _sc_agg_cache[e_pt](xv, gidx, ddst)
    return out.reshape(NC, N_ACC, 128)


def _make_tc_layer(d_in: int, split_features: bool):
    R = 1000
    grid = N_NODES // R

    def body(x_ref, p_ref, w_ref, b_ref, g_ref, beta_ref, o_ref):
        x = x_ref[...]
        if split_features:
            h = x + jnp.concatenate([p_ref[0], p_ref[1]], axis=1)
        else:
            h = x + p_ref[0] + p_ref[1]
        z = jnp.dot(h, w_ref[...], preferred_element_type=jnp.float32)
        z = z + b_ref[...]
        mu = jnp.mean(z, axis=1, keepdims=True)
        zc = z - mu
        var = jnp.mean(zc * zc, axis=1, keepdims=True)
        zn = zc * lax.rsqrt(var + LN_EPS)
        h = zn * g_ref[...] + beta_ref[...]
        o_ref[...] = jnp.maximum(h, 0.0)

    return pl.pallas_call(
        body,
        grid=(grid,),
        in_specs=[
            pl.BlockSpec((R, d_in), lambda i: (i, 0)),
            pl.BlockSpec((NC, R, 128), lambda i: (0, i, 0)),

            pl.BlockSpec((d_in, HID), lambda i: (0, 0)),
            pl.BlockSpec((1, HID), lambda i: (0, 0)),
            pl.BlockSpec((1, HID), lambda i: (0, 0)),
            pl.BlockSpec((1, HID), lambda i: (0, 0)),
        ],
        out_specs=pl.BlockSpec((R, HID), lambda i: (i, 0)),
        out_shape=jax.ShapeDtypeStruct((N_NODES, HID), jnp.float32),
    )


_tc_layer_cache = {}


def _tc_layer(x, parts, W, b, g, beta, split_features):
    key = (x.shape[1], split_features)
    if key not in _tc_layer_cache:
        _tc_layer_cache[key] = _make_tc_layer(x.shape[1], split_features)
    return _tc_layer_cache[key](
        x, parts, W, b.reshape(1, HID), g.reshape(1, HID), beta.reshape(1, HID))


def _edge_layout_split(src, dst):
    per_tile = N_EDGES // (NC * NS)
    ppt = _pad_per_tile(per_tile)
    s2 = src.reshape(NC * NS, per_tile)
    d2 = dst.reshape(NC * NS, per_tile)
    pad = ((0, 0), (0, ppt - per_tile))
    gidx = jnp.pad(s2, pad, constant_values=0).reshape(-1)
    ddst = jnp.pad(d2, pad, constant_values=DUMMY_ROW).reshape(-1)
    return gidx, ddst, ppt


def _edge_layout_full(src, dst):
    per_tile = N_EDGES // NS
    ppt = _pad_per_tile(per_tile)
    s2 = src.reshape(NS, per_tile)
    d2 = dst.reshape(NS, per_tile)
    pad = ((0, 0), (0, ppt - per_tile))
    s2 = jnp.pad(s2, pad, constant_values=0)
    d2 = jnp.pad(d2, pad, constant_values=DUMMY_ROW)
    gidx = jnp.stack([2 * s2, 2 * s2 + 1]).reshape(-1)
    ddst = jnp.stack([d2, d2]).reshape(-1)
    return gidx, ddst, ppt


def kernel(x, edge_index, W0, b0, g0, beta0, W1, b1, g1, beta1, W2, b2, g2, beta2):
    src = edge_index[0].astype(jnp.int32)
    dst = edge_index[1].astype(jnp.int32)

    gidx0, ddst0, ppt0 = _edge_layout_split(src, dst)
    gidx1, ddst1, ppt1 = _edge_layout_full(src, dst)

    p0 = _sc_aggregate(x, gidx0, ddst0, ppt0)
    h1 = _tc_layer(x, p0, W0, b0, g0, beta0, split_features=False)

    p1 = _sc_aggregate(h1.reshape(2 * N_NODES, 128), gidx1, ddst1, ppt1)
    h2 = _tc_layer(h1, p1, W1, b1, g1, beta1, split_features=True)

    p2 = _sc_aggregate(h2.reshape(2 * N_NODES, 128), gidx1, ddst1, ppt1)
    h3 = _tc_layer(h2, p2, W2, b2, g2, beta2, split_features=True)

    return jnp.concatenate([x, h1, h2, h3], axis=1)
